# Initial kernel scaffold; baseline (speedup 1.0000x reference)
import jax
import jax.numpy as jnp
from jax.experimental import pallas as pl


def kernel(hidden_states, gate_w, w1, w3, w2, sw1, sw3, sw2):
    raise NotImplementedError("write your pallas kernel here")


# speedup vs baseline: 2.2933x; 2.2933x over previous
import functools

import jax
import jax.numpy as jnp
from jax.experimental import pallas as pl
from jax.experimental.pallas import tpu as pltpu

T = 2048
D = 1024
E = 8
DF = 512
CAP = 512
_NEG = -1e30


def _routing_body(x_ref, gw_ref, comb_ref):
    x = x_ref[...]
    gw = gw_ref[...]
    logits = jax.lax.dot_general(
        x, gw, (((1,), (1,)), ((), ())), preferred_element_type=jnp.float32
    )
    m = jnp.max(logits, axis=-1, keepdims=True)
    ex = jnp.exp(logits - m)
    scores = ex / jnp.sum(ex, axis=-1, keepdims=True)
    eidx = jax.lax.broadcasted_iota(jnp.int32, (T, E), 1)
    s0 = jnp.max(scores, axis=-1, keepdims=True)
    i0 = jnp.min(jnp.where(scores >= s0, eidx, E), axis=-1, keepdims=True)
    oh0 = eidx == i0
    sc1 = jnp.where(oh0, _NEG, scores)
    s1 = jnp.max(sc1, axis=-1, keepdims=True)
    i1 = jnp.min(jnp.where(sc1 >= s1, eidx, E), axis=-1, keepdims=True)
    oh1 = eidx == i1
    cnt = oh0.astype(jnp.bfloat16) + oh1.astype(jnp.bfloat16)
    r = jax.lax.broadcasted_iota(jnp.int32, (T, T), 0)
    c = jax.lax.broadcasted_iota(jnp.int32, (T, T), 1)
    lmask = (c < r).astype(jnp.bfloat16)
    cum = jax.lax.dot_general(
        lmask, cnt, (((1,), (0,)), ((), ())), preferred_element_type=jnp.float32
    )
    pos0 = jnp.sum(jnp.where(oh0, cum, 0.0), axis=-1, keepdims=True)
    pos1 = jnp.sum(jnp.where(oh1, cum, 0.0), axis=-1, keepdims=True)
    v0 = pos0 < CAP
    v1 = pos1 < CAP
    denom = s0 + s1 + 1e-20
    w0 = jnp.where(v0, s0 / denom, 0.0)
    w1 = jnp.where(v1, s1 / denom, 0.0)
    comb_ref[...] = jnp.where(oh0, w0, 0.0) + jnp.where(oh1, w1, 0.0)


def _routing(x, gate_w, interpret=False):
    return pl.pallas_call(
        _routing_body,
        out_shape=jax.ShapeDtypeStruct((T, E), jnp.float32),
        interpret=interpret,
    )(x, gate_w)


def _silu(h):
    return h / (1.0 + jnp.exp(-h))


def _moe_body(comb_ref, x_ref, w1_ref, w3_ref, w2_ref, sw1_ref, sw3_ref, sw2_ref, y_ref):
    e = pl.program_id(0)
    xb = x_ref[...].astype(jnp.bfloat16)
    a = w1_ref[0].astype(jnp.bfloat16)
    b = w3_ref[0].astype(jnp.bfloat16)
    cw = w2_ref[0].astype(jnp.bfloat16)
    h1 = jax.lax.dot_general(
        xb, a, (((1,), (1,)), ((), ())), preferred_element_type=jnp.float32
    )
    h3 = jax.lax.dot_general(
        xb, b, (((1,), (1,)), ((), ())), preferred_element_type=jnp.float32
    )
    h = (_silu(h1) * h3).astype(jnp.bfloat16)
    out_e = jax.lax.dot_general(
        h, cw, (((1,), (1,)), ((), ())), preferred_element_type=jnp.float32
    )
    lane = jax.lax.broadcasted_iota(jnp.int32, (T, E), 1)
    cvec = jnp.sum(jnp.where(lane == e, comb_ref[...], 0.0), axis=-1, keepdims=True)
    contrib = cvec * out_e

    @pl.when(e == 0)
    def _():
        y_ref[...] = contrib

    @pl.when(e > 0)
    def _():
        y_ref[...] += contrib

    tb = T // E
    rs = e * tb
    xs = x_ref[pl.ds(rs, tb), :].astype(jnp.bfloat16)
    sa = sw1_ref[...].astype(jnp.bfloat16)
    sb = sw3_ref[...].astype(jnp.bfloat16)
    sc = sw2_ref[...].astype(jnp.bfloat16)
    sh1 = jax.lax.dot_general(
        xs, sa, (((1,), (1,)), ((), ())), preferred_element_type=jnp.float32
    )
    sh3 = jax.lax.dot_general(
        xs, sb, (((1,), (1,)), ((), ())), preferred_element_type=jnp.float32
    )
    hs = (_silu(sh1) * sh3).astype(jnp.bfloat16)
    out_s = jax.lax.dot_general(
        hs, sc, (((1,), (1,)), ((), ())), preferred_element_type=jnp.float32
    )
    y_ref[pl.ds(rs, tb), :] += out_s


def _moe(comb, x, w1, w3, w2, sw1, sw3, sw2, interpret=False):
    grid = (E,)
    return pl.pallas_call(
        _moe_body,
        grid=grid,
        in_specs=[
            pl.BlockSpec((T, E), lambda e: (0, 0)),
            pl.BlockSpec((T, D), lambda e: (0, 0)),
            pl.BlockSpec((1, DF, D), lambda e: (e, 0, 0)),
            pl.BlockSpec((1, DF, D), lambda e: (e, 0, 0)),
            pl.BlockSpec((1, D, DF), lambda e: (e, 0, 0)),
            pl.BlockSpec((2 * DF, D), lambda e: (0, 0)),
            pl.BlockSpec((2 * DF, D), lambda e: (0, 0)),
            pl.BlockSpec((D, 2 * DF), lambda e: (0, 0)),
        ],
        out_specs=pl.BlockSpec((T, D), lambda e: (0, 0)),
        out_shape=jax.ShapeDtypeStruct((T, D), jnp.float32),
        compiler_params=pltpu.CompilerParams(
            dimension_semantics=("arbitrary",),
        ),
        interpret=interpret,
    )(comb, x, w1, w3, w2, sw1, sw3, sw2)


def kernel(hidden_states, gate_w, w1, w3, w2, sw1, sw3, sw2):
    orig_shape = hidden_states.shape
    x = hidden_states.reshape(T, D).astype(jnp.float32)
    comb = _routing(x, gate_w.astype(jnp.float32))
    y = _moe(comb, x, w1, w3, w2, sw1, sw3, sw2)
    return y.reshape(orig_shape)
